# baseline (speedup 1.0000x reference)
import functools

import numpy as np
import jax
import jax.numpy as jnp
from jax import lax
from jax.experimental import pallas as pl
from jax.experimental.pallas import tpu as pltpu
from jax.experimental.pallas import tpu_sc as plsc

M = 512
N = 64
P = 16384
CF = 130
NPTS = 512
FG = 32
BG = 32
R = 64
TWO_PI = float(2 * np.pi)


def _iou_terms(rx, ry_, rz, rh, rw, rl, gx, gy, gz, gh, gw, gl):
    ix1 = jnp.maximum(rx - rl / 2, gx - gl / 2)
    ix2 = jnp.minimum(rx + rl / 2, gx + gl / 2)
    iz1 = jnp.maximum(rz - rw / 2, gz - gw / 2)
    iz2 = jnp.minimum(rz + rw / 2, gz + gw / 2)
    iy1 = jnp.maximum(ry_ - rh, gy - gh)
    iy2 = jnp.minimum(ry_, gy)
    inter = (jnp.clip(ix2 - ix1, 0.0) * jnp.clip(iz2 - iz1, 0.0)
             * jnp.clip(iy2 - iy1, 0.0))
    vol_r = rh * rw * rl
    vol_g = gh * gw * gl
    return inter / jnp.maximum(vol_r + vol_g - inter, 1e-6)


def _select_body(rois_ref, rois_t_ref, gts_ref, rois_out_ref, gt_out_ref,
                 prm_ref):
    r = rois_ref[0]
    rT = rois_t_ref[0]
    g = gts_ref[0]

    def gcol(c):
        return lax.broadcast_in_dim(g[:, c], (M, N), (1,))

    iou = _iou_terms(*[r[:, c:c + 1] for c in range(6)],
                     *[gcol(c) for c in range(6)])
    iouT = _iou_terms(*[rT[c:c + 1] for c in range(6)],
                      *[g[:, c:c + 1] for c in range(6)])

    max_ov = jnp.max(iou, axis=1, keepdims=True)
    maxT = jnp.max(iouT, axis=0, keepdims=True)
    jN = lax.broadcasted_iota(jnp.int32, (M, N), 1)
    gt_asg = jnp.min(jnp.where(iou == max_ov, jN, N), axis=1, keepdims=True)

    vT = jnp.broadcast_to(maxT, (M, M))
    iI = lax.broadcasted_iota(jnp.int32, (M, M), 0)
    jJ = lax.broadcasted_iota(jnp.int32, (M, M), 1)
    before = (vT > max_ov) | ((vT == max_ov) & (jJ < iI))
    rank = jnp.sum(before.astype(jnp.int32), axis=1, keepdims=True)

    slot = jnp.where(rank < FG, rank,
                     jnp.where(rank >= M - BG, rank - (M - R), -1))
    sT = lax.broadcasted_iota(jnp.int32, (M, R), 1)
    O = (slot == sT).astype(jnp.float32)

    dn = (((0,), (0,)), ((), ()))
    hp = lax.Precision.HIGHEST
    rois_sel = lax.dot_general(r, O, dn, precision=hp)
    rois_out_ref[0] = rois_sel

    jNg = lax.broadcasted_iota(jnp.int32, (M, N), 1)
    OG = (gt_asg == jNg).astype(jnp.float32)
    gpick = jnp.dot(OG, g, precision=hp)
    gt_sel = lax.dot_general(gpick, O, dn, precision=hp)
    iou_sel = lax.dot_general(max_ov, O, dn, precision=hp)

    rsel = [rois_sel[c:c + 1] for c in range(7)]
    gsel = [gt_sel[c:c + 1] for c in range(7)]
    cx, cy, cz, h, w, l, ry = rsel
    rym = jnp.mod(ry, TWO_PI)
    cosr = jnp.cos(ry)
    sinr = jnp.sin(ry)
    cosm = jnp.cos(rym)
    sinm = jnp.sin(rym)

    gxl = gsel[0] - cx
    gyl = gsel[1] - cy
    gzl = gsel[2] - cz
    gxn = cosm * gxl - sinm * gzl
    gzn = sinm * gxl + cosm * gzl
    gt_ry_o = gsel[6] - rym
    gt_out_ref[0] = jnp.concatenate(
        [gxn, gyl, gzn, gsel[3], gsel[4], gsel[5], gt_ry_o,
         jnp.zeros((1, R), jnp.float32)], axis=0)

    invalid = (iou_sel > 0.45) & (iou_sel < 0.6)
    clsv = jnp.where(invalid, -1.0, (iou_sel > 0.6).astype(jnp.float32))
    regv = (iou_sel > 0.55).astype(jnp.float32)
    xlim = l / 2 + 1.0
    zlim = w / 2 + 1.0
    ylo = -h - 1.0
    pad = jnp.zeros((5, R), jnp.float32)
    prm_ref[0] = jnp.concatenate(
        [cx, cy, cz, cosr, sinr, xlim, zlim, ylo, clsv, regv, iou_sel, pad],
        axis=0)


def _stage1(rois_p, rois_t, gts_p):
    B = rois_p.shape[0]
    return pl.pallas_call(
        _select_body,
        grid=(B,),
        in_specs=[
            pl.BlockSpec((1, M, 8), lambda b: (b, 0, 0)),
            pl.BlockSpec((1, 8, M), lambda b: (b, 0, 0)),
            pl.BlockSpec((1, N, 8), lambda b: (b, 0, 0)),
        ],
        out_specs=[
            pl.BlockSpec((1, 8, R), lambda b: (b, 0, 0)),
            pl.BlockSpec((1, 8, R), lambda b: (b, 0, 0)),
            pl.BlockSpec((1, 16, R), lambda b: (b, 0, 0)),
        ],
        out_shape=[
            jax.ShapeDtypeStruct((B, 8, R), jnp.float32),
            jax.ShapeDtypeStruct((B, 8, R), jnp.float32),
            jax.ShapeDtypeStruct((B, 16, R), jnp.float32),
        ],
    )(rois_p, rois_t, gts_p)


_info = plsc.get_sparse_core_info()
_NC, _NS, _L = _info.num_cores, _info.num_subcores, _info.num_lanes
_NW = _NC * _NS
_RPW = (4 * R) // _NW
_WPB = R // _RPW
_NSTEP = P // 16
_NCH = NPTS // 128


def _pool_body(xyzT, depth, seg, table, prm_hbm,
               pts_out, feat_out, sd_out, lab_out,
               Xb, Yb, Zb, Db, Sb, bufA, bufB, idxb, ptst, sdst, fst, prmv,
               labacc, sem1, sem2):
    wid = lax.axis_index("s") * _NC + lax.axis_index("c")
    b = wid // _WPB
    grp = wid % _WPB
    bP = b * P

    pltpu.sync_copy(xyzT.at[pl.ds((b * 3 + 0) * P, P)], Xb)
    pltpu.sync_copy(xyzT.at[pl.ds((b * 3 + 1) * P, P)], Yb)
    pltpu.sync_copy(xyzT.at[pl.ds((b * 3 + 2) * P, P)], Zb)
    pltpu.sync_copy(depth.at[pl.ds(bP, P)], Db)
    pltpu.sync_copy(seg.at[pl.ds(bP, P)], Sb)
    pltpu.sync_copy(prm_hbm.at[pl.ds(wid * 128, 128)], prmv)

    lanes = lax.iota(jnp.int32, 16)

    def roi_body(k, carry):
        rb = grp * _RPW + k
        gidx = b * R + rb
        pv = prmv[pl.ds(k * 16, 16)]

        def getp(j):
            return jnp.sum(jnp.where(lanes == j, pv, 0.0))

        cx = getp(0)
        cy = getp(1)
        cz = getp(2)
        cosr = getp(3)
        sinr = getp(4)
        xlim = getp(5)
        zlim = getp(6)
        ylo = getp(7)
        clsv = getp(8)
        regv = getp(9)
        aco = jnp.abs(cosr)
        asi = jnp.abs(sinr)
        xb = xlim * aco + zlim * asi
        zb = xlim * asi + zlim * aco

        def mstep(j, offA):
            base = j * 16
            xv = Xb[pl.ds(base, 16)]
            yv = Yb[pl.ds(base, 16)]
            zv = Zb[pl.ds(base, 16)]
            lx = xv - cx
            ly = yv - cy
            lz = zv - cz
            ym = (ly > ylo) & (ly < 1.0)
            pre = (jnp.abs(lx) < xb) & (jnp.abs(lz) < zb) & ym
            offB = base - offA
            cand = jnp.any(pre) | (offB < NPTS)

            def full(o):
                xn = cosr * lx - sinr * lz
                zn = sinr * lx + cosr * lz
                m = (jnp.abs(xn) < xlim) & (jnp.abs(zn) < zlim) & ym
                pidx = base + lanes

                @pl.when(o < NPTS)
                def _():
                    plsc.store_compressed(bufA.at[pl.ds(o, 16)], pidx, mask=m)

                @pl.when(offB < NPTS)
                def _():
                    plsc.store_compressed(bufB.at[pl.ds(offB, 16)], pidx,
                                          mask=jnp.logical_not(m))

                return o + jnp.sum(m.astype(jnp.int32))

            return lax.cond(cand, full, lambda o: o, offA)

        n = lax.fori_loop(0, _NSTEP, mstep, jnp.int32(0))
        nsel = jnp.minimum(n, NPTS)

        zeros16 = jnp.zeros((16,), jnp.int32)
        for j in range(NPTS // 16):
            p = j * 16 + lanes
            av = plsc.load_gather(bufA, [p])
            bv = plsc.load_gather(bufB, [jnp.maximum(p - nsel, 0)])
            iv = jnp.where(p < nsel, av, bv)
            idxb[pl.ds(j * 16, 16)] = iv + bP
            gx = plsc.load_gather(Xb, [iv])
            gy = plsc.load_gather(Yb, [iv])
            gz = plsc.load_gather(Zb, [iv])
            sv = plsc.load_gather(Sb, [iv])
            dv = plsc.load_gather(Db, [iv])
            lx = gx - cx
            lyy = gy - cy
            lz = gz - cz
            xn = cosr * lx - sinr * lz
            zn = sinr * lx + cosr * lz
            p3 = p * 3
            plsc.store_scatter(ptst, [p3], xn)
            plsc.store_scatter(ptst, [p3 + 1], lyy)
            plsc.store_scatter(ptst, [p3 + 2], zn)
            p2 = p * 2
            plsc.store_scatter(sdst, [p2], sv)
            plsc.store_scatter(sdst, [p2 + 1], dv / 70.0 - 0.5)
        pltpu.sync_copy(ptst, pts_out.at[pl.ds(gidx * (NPTS * 3), NPTS * 3)])
        pltpu.sync_copy(sdst, sd_out.at[pl.ds(gidx * (NPTS * 2), NPTS * 2)])

        def gather(ch, sb):
            return pltpu.async_copy(table.at[idxb.at[pl.ds(ch * 128, 128)]],
                                    fst.at[pl.ds(sb * 128, 128)], sem1)

        def putout(ch, sb):
            return pltpu.async_copy(
                fst.at[pl.ds(sb * 128, 128)],
                feat_out.at[pl.ds(gidx * NPTS + ch * 128, 128)], sem2)

        g0 = gather(0, 0)
        g0.wait()
        c0 = putout(0, 0)
        g1 = gather(1, 1)
        g1.wait()
        c1 = putout(1, 1)
        c0.wait()
        g2 = gather(2, 0)
        g2.wait()
        c2 = putout(2, 0)
        c1.wait()
        g3 = gather(3, 1)
        g3.wait()
        c3 = putout(3, 1)
        c2.wait()
        c3.wait()

        valid = n > 0
        cls_i = jnp.where(valid, clsv.astype(jnp.int32), -1)
        reg_i = jnp.where(valid, regv.astype(jnp.int32), 0)
        lane0 = lanes == 0
        plsc.store_scatter(labacc, [zeros16 + k],
                           jnp.broadcast_to(cls_i, (16,)), mask=lane0)
        plsc.store_scatter(labacc, [zeros16 + (k + _RPW)],
                           jnp.broadcast_to(reg_i, (16,)), mask=lane0)
        return carry

    lax.fori_loop(0, _RPW, roi_body, jnp.int32(0))
    pltpu.sync_copy(labacc, lab_out.at[pl.ds(wid * 128, 128)])


def _stage2(xyzT, depth, seg, table, prm):
    B = depth.shape[0] // P
    nroi = B * R
    mesh = plsc.VectorSubcoreMesh(core_axis_name="c", subcore_axis_name="s")
    fn = functools.partial(
        pl.kernel,
        mesh=mesh,
        compiler_params=pltpu.CompilerParams(needs_layout_passes=False),
        out_type=[
            jax.ShapeDtypeStruct((nroi * NPTS * 3,), jnp.float32),
            jax.ShapeDtypeStruct((nroi * NPTS, 128), jnp.float32),
            jax.ShapeDtypeStruct((nroi * NPTS * 2,), jnp.float32),
            jax.ShapeDtypeStruct((_NW * 128,), jnp.int32),
        ],
        scratch_types=[
            pltpu.VMEM((P,), jnp.float32),
            pltpu.VMEM((P,), jnp.float32),
            pltpu.VMEM((P,), jnp.float32),
            pltpu.VMEM((P,), jnp.float32),
            pltpu.VMEM((P,), jnp.float32),
            pltpu.VMEM((NPTS + 16,), jnp.int32),
            pltpu.VMEM((NPTS + 16,), jnp.int32),
            pltpu.VMEM((NPTS,), jnp.int32),
            pltpu.VMEM((NPTS * 3,), jnp.float32),
            pltpu.VMEM((NPTS * 2,), jnp.float32),
            pltpu.VMEM((2 * 128, 128), jnp.float32),
            pltpu.VMEM((128,), jnp.float32),
            pltpu.VMEM((128,), jnp.int32),
            pltpu.SemaphoreType.DMA,
            pltpu.SemaphoreType.DMA,
        ],
    )(_pool_body)
    return fn(xyzT, depth, seg, table, prm)


def kernel(roi_boxes3d, gt_boxes3d, rpn_xyz, rpn_features, seg_mask, pts_depth):
    B = roi_boxes3d.shape[0]
    rois_p = jnp.pad(roi_boxes3d, ((0, 0), (0, 0), (0, 1)))
    rois_t = rois_p.transpose(0, 2, 1)
    rois_sel_t, gt_out_t, prm_t = _stage1(rois_p, rois_t, gt_boxes3d)
    rois_sel = rois_sel_t.transpose(0, 2, 1)
    gt_out = gt_out_t.transpose(0, 2, 1)
    prm = prm_t.transpose(0, 2, 1)

    xyzT = rpn_xyz.transpose(0, 2, 1).reshape(-1)
    table = rpn_features.reshape(B * P, 128)

    pts, feats, sd, lab = _stage2(
        xyzT, pts_depth.reshape(-1), seg_mask.reshape(-1), table,
        prm.reshape(-1))
    lab2 = lab.reshape(_NW, 128)
    cls_label = lab2[:, :_RPW].reshape(-1)
    reg_valid = lab2[:, _RPW:2 * _RPW].reshape(-1)

    feats_full = jnp.concatenate(
        [sd.reshape(B * R, NPTS, 2), feats.reshape(B * R, NPTS, 128)],
        axis=-1)
    return (pts.reshape(B * R, NPTS, 3),
            feats_full,
            cls_label,
            reg_valid,
            gt_out[..., :7].reshape(B * R, 7),
            prm[..., 10].reshape(B * R),
            rois_sel[..., :7].reshape(B * R, 7))


# speedup vs baseline: 1.1970x; 1.1970x over previous
import functools

import numpy as np
import jax
import jax.numpy as jnp
from jax import lax
from jax.experimental import pallas as pl
from jax.experimental.pallas import tpu as pltpu
from jax.experimental.pallas import tpu_sc as plsc

M = 512
N = 64
P = 16384
CF = 130
NPTS = 512
FG = 32
BG = 32
R = 64
TWO_PI = float(2 * np.pi)


def _iou_terms(rx, ry_, rz, rh, rw, rl, gx, gy, gz, gh, gw, gl):
    ix1 = jnp.maximum(rx - rl / 2, gx - gl / 2)
    ix2 = jnp.minimum(rx + rl / 2, gx + gl / 2)
    iz1 = jnp.maximum(rz - rw / 2, gz - gw / 2)
    iz2 = jnp.minimum(rz + rw / 2, gz + gw / 2)
    iy1 = jnp.maximum(ry_ - rh, gy - gh)
    iy2 = jnp.minimum(ry_, gy)
    inter = (jnp.clip(ix2 - ix1, 0.0) * jnp.clip(iz2 - iz1, 0.0)
             * jnp.clip(iy2 - iy1, 0.0))
    vol_r = rh * rw * rl
    vol_g = gh * gw * gl
    return inter / jnp.maximum(vol_r + vol_g - inter, 1e-6)


def _select_body(rois_ref, rois_t_ref, gts_ref, rois_out_ref, gt_out_ref,
                 prm_ref):
    r = rois_ref[0]
    rT = rois_t_ref[0]
    g = gts_ref[0]

    def gcol(c):
        return lax.broadcast_in_dim(g[:, c], (M, N), (1,))

    iou = _iou_terms(*[r[:, c:c + 1] for c in range(6)],
                     *[gcol(c) for c in range(6)])
    iouT = _iou_terms(*[rT[c:c + 1] for c in range(6)],
                      *[g[:, c:c + 1] for c in range(6)])

    max_ov = jnp.max(iou, axis=1, keepdims=True)
    maxT = jnp.max(iouT, axis=0, keepdims=True)
    jN = lax.broadcasted_iota(jnp.int32, (M, N), 1)
    gt_asg = jnp.min(jnp.where(iou == max_ov, jN, N), axis=1, keepdims=True)

    vT = jnp.broadcast_to(maxT, (M, M))
    iI = lax.broadcasted_iota(jnp.int32, (M, M), 0)
    jJ = lax.broadcasted_iota(jnp.int32, (M, M), 1)
    before = (vT > max_ov) | ((vT == max_ov) & (jJ < iI))
    rank = jnp.sum(before.astype(jnp.int32), axis=1, keepdims=True)

    slot = jnp.where(rank < FG, rank,
                     jnp.where(rank >= M - BG, rank - (M - R), -1))
    sT = lax.broadcasted_iota(jnp.int32, (M, R), 1)
    O = (slot == sT).astype(jnp.float32)

    dn = (((0,), (0,)), ((), ()))
    hp = lax.Precision.HIGHEST
    rois_sel = lax.dot_general(r, O, dn, precision=hp)
    rois_out_ref[0] = rois_sel

    jNg = lax.broadcasted_iota(jnp.int32, (M, N), 1)
    OG = (gt_asg == jNg).astype(jnp.float32)
    gpick = jnp.dot(OG, g, precision=hp)
    gt_sel = lax.dot_general(gpick, O, dn, precision=hp)
    iou_sel = lax.dot_general(max_ov, O, dn, precision=hp)

    rsel = [rois_sel[c:c + 1] for c in range(7)]
    gsel = [gt_sel[c:c + 1] for c in range(7)]
    cx, cy, cz, h, w, l, ry = rsel
    rym = jnp.mod(ry, TWO_PI)
    cosr = jnp.cos(ry)
    sinr = jnp.sin(ry)
    cosm = jnp.cos(rym)
    sinm = jnp.sin(rym)

    gxl = gsel[0] - cx
    gyl = gsel[1] - cy
    gzl = gsel[2] - cz
    gxn = cosm * gxl - sinm * gzl
    gzn = sinm * gxl + cosm * gzl
    gt_ry_o = gsel[6] - rym
    gt_out_ref[0] = jnp.concatenate(
        [gxn, gyl, gzn, gsel[3], gsel[4], gsel[5], gt_ry_o,
         jnp.zeros((1, R), jnp.float32)], axis=0)

    invalid = (iou_sel > 0.45) & (iou_sel < 0.6)
    clsv = jnp.where(invalid, -1.0, (iou_sel > 0.6).astype(jnp.float32))
    regv = (iou_sel > 0.55).astype(jnp.float32)
    xlim = l / 2 + 1.0
    zlim = w / 2 + 1.0
    ylo = -h - 1.0
    pad = jnp.zeros((5, R), jnp.float32)
    prm_ref[0] = jnp.concatenate(
        [cx, cy, cz, cosr, sinr, xlim, zlim, ylo, clsv, regv, iou_sel, pad],
        axis=0)


def _stage1(rois_p, rois_t, gts_p):
    B = rois_p.shape[0]
    return pl.pallas_call(
        _select_body,
        grid=(B,),
        in_specs=[
            pl.BlockSpec((1, M, 8), lambda b: (b, 0, 0)),
            pl.BlockSpec((1, 8, M), lambda b: (b, 0, 0)),
            pl.BlockSpec((1, N, 8), lambda b: (b, 0, 0)),
        ],
        out_specs=[
            pl.BlockSpec((1, 8, R), lambda b: (b, 0, 0)),
            pl.BlockSpec((1, 8, R), lambda b: (b, 0, 0)),
            pl.BlockSpec((1, 16, R), lambda b: (b, 0, 0)),
        ],
        out_shape=[
            jax.ShapeDtypeStruct((B, 8, R), jnp.float32),
            jax.ShapeDtypeStruct((B, 8, R), jnp.float32),
            jax.ShapeDtypeStruct((B, 16, R), jnp.float32),
        ],
    )(rois_p, rois_t, gts_p)


_info = plsc.get_sparse_core_info()
_NC, _NS, _L = _info.num_cores, _info.num_subcores, _info.num_lanes
_NW = _NC * _NS
_RPW = (4 * R) // _NW
_WPB = R // _RPW
_NSTEP = P // 16
_NCH = NPTS // 128


def _pool_body(xyzT, depth, seg, table, prm_hbm,
               pts_out, feat_out, sd_out, lab_out,
               Xb, Yb, Zb, Db, Sb, bufA, bufB, idxb, ptst, sdst, fst, prmv,
               labacc, sem1, sem2):
    wid = lax.axis_index("s") * _NC + lax.axis_index("c")
    b = wid // _WPB
    grp = wid % _WPB
    bP = b * P

    pltpu.sync_copy(xyzT.at[pl.ds((b * 3 + 0) * P, P)], Xb)
    pltpu.sync_copy(xyzT.at[pl.ds((b * 3 + 1) * P, P)], Yb)
    pltpu.sync_copy(xyzT.at[pl.ds((b * 3 + 2) * P, P)], Zb)
    pltpu.sync_copy(depth.at[pl.ds(bP, P)], Db)
    pltpu.sync_copy(seg.at[pl.ds(bP, P)], Sb)
    pltpu.sync_copy(prm_hbm.at[pl.ds(wid * 128, 128)], prmv)

    lanes = lax.iota(jnp.int32, 16)

    def roi_body(k, carry):
        rb = grp * _RPW + k
        gidx = b * R + rb
        pv = prmv[pl.ds(k * 16, 16)]

        def getp(j):
            return jnp.sum(jnp.where(lanes == j, pv, 0.0))

        cx = getp(0)
        cy = getp(1)
        cz = getp(2)
        cosr = getp(3)
        sinr = getp(4)
        xlim = getp(5)
        zlim = getp(6)
        ylo = getp(7)
        clsv = getp(8)
        regv = getp(9)
        aco = jnp.abs(cosr)
        asi = jnp.abs(sinr)
        xb = xlim * aco + zlim * asi
        zb = xlim * asi + zlim * aco

        def mstep(j, offA):
            base = j * 16
            xv = Xb[pl.ds(base, 16)]
            yv = Yb[pl.ds(base, 16)]
            zv = Zb[pl.ds(base, 16)]
            lx = xv - cx
            ly = yv - cy
            lz = zv - cz
            xn = cosr * lx - sinr * lz
            zn = sinr * lx + cosr * lz
            m = ((jnp.abs(xn) < xlim) & (jnp.abs(zn) < zlim)
                 & (ly > ylo) & (ly < 1.0))
            pidx = base + lanes
            plsc.store_compressed(bufA.at[pl.ds(jnp.minimum(offA, NPTS), 16)],
                                  pidx, mask=m)
            offB = jnp.minimum(base - offA, NPTS)
            plsc.store_compressed(bufB.at[pl.ds(offB, 16)], pidx,
                                  mask=jnp.logical_not(m))
            return offA + jnp.sum(m.astype(jnp.int32))

        n = lax.fori_loop(0, _NSTEP, mstep, jnp.int32(0))
        nsel = jnp.minimum(n, NPTS)

        zeros16 = jnp.zeros((16,), jnp.int32)
        for j in range(NPTS // 16):
            p = j * 16 + lanes
            av = plsc.load_gather(bufA, [p])
            bv = plsc.load_gather(bufB, [jnp.maximum(p - nsel, 0)])
            iv = jnp.where(p < nsel, av, bv)
            idxb[pl.ds(j * 16, 16)] = iv + bP
            gx = plsc.load_gather(Xb, [iv])
            gy = plsc.load_gather(Yb, [iv])
            gz = plsc.load_gather(Zb, [iv])
            sv = plsc.load_gather(Sb, [iv])
            dv = plsc.load_gather(Db, [iv])
            lx = gx - cx
            lyy = gy - cy
            lz = gz - cz
            xn = cosr * lx - sinr * lz
            zn = sinr * lx + cosr * lz
            p3 = p * 3
            plsc.store_scatter(ptst, [p3], xn)
            plsc.store_scatter(ptst, [p3 + 1], lyy)
            plsc.store_scatter(ptst, [p3 + 2], zn)
            p2 = p * 2
            plsc.store_scatter(sdst, [p2], sv)
            plsc.store_scatter(sdst, [p2 + 1], dv / 70.0 - 0.5)
        pltpu.sync_copy(ptst, pts_out.at[pl.ds(gidx * (NPTS * 3), NPTS * 3)])
        pltpu.sync_copy(sdst, sd_out.at[pl.ds(gidx * (NPTS * 2), NPTS * 2)])

        def gather(ch, sb):
            return pltpu.async_copy(table.at[idxb.at[pl.ds(ch * 128, 128)]],
                                    fst.at[pl.ds(sb * 128, 128)], sem1)

        def putout(ch, sb):
            return pltpu.async_copy(
                fst.at[pl.ds(sb * 128, 128)],
                feat_out.at[pl.ds(gidx * NPTS + ch * 128, 128)], sem2)

        g0 = gather(0, 0)
        g0.wait()
        c0 = putout(0, 0)
        g1 = gather(1, 1)
        g1.wait()
        c1 = putout(1, 1)
        c0.wait()
        g2 = gather(2, 0)
        g2.wait()
        c2 = putout(2, 0)
        c1.wait()
        g3 = gather(3, 1)
        g3.wait()
        c3 = putout(3, 1)
        c2.wait()
        c3.wait()

        valid = n > 0
        cls_i = jnp.where(valid, clsv.astype(jnp.int32), -1)
        reg_i = jnp.where(valid, regv.astype(jnp.int32), 0)
        lane0 = lanes == 0
        plsc.store_scatter(labacc, [zeros16 + k],
                           jnp.broadcast_to(cls_i, (16,)), mask=lane0)
        plsc.store_scatter(labacc, [zeros16 + (k + _RPW)],
                           jnp.broadcast_to(reg_i, (16,)), mask=lane0)
        return carry

    lax.fori_loop(0, _RPW, roi_body, jnp.int32(0))
    pltpu.sync_copy(labacc, lab_out.at[pl.ds(wid * 128, 128)])


def _stage2(xyzT, depth, seg, table, prm):
    B = depth.shape[0] // P
    nroi = B * R
    mesh = plsc.VectorSubcoreMesh(core_axis_name="c", subcore_axis_name="s")
    fn = functools.partial(
        pl.kernel,
        mesh=mesh,
        compiler_params=pltpu.CompilerParams(needs_layout_passes=False),
        out_type=[
            jax.ShapeDtypeStruct((nroi * NPTS * 3,), jnp.float32),
            jax.ShapeDtypeStruct((nroi * NPTS, 128), jnp.float32),
            jax.ShapeDtypeStruct((nroi * NPTS * 2,), jnp.float32),
            jax.ShapeDtypeStruct((_NW * 128,), jnp.int32),
        ],
        scratch_types=[
            pltpu.VMEM((P,), jnp.float32),
            pltpu.VMEM((P,), jnp.float32),
            pltpu.VMEM((P,), jnp.float32),
            pltpu.VMEM((P,), jnp.float32),
            pltpu.VMEM((P,), jnp.float32),
            pltpu.VMEM((NPTS + 16,), jnp.int32),
            pltpu.VMEM((NPTS + 16,), jnp.int32),
            pltpu.VMEM((NPTS,), jnp.int32),
            pltpu.VMEM((NPTS * 3,), jnp.float32),
            pltpu.VMEM((NPTS * 2,), jnp.float32),
            pltpu.VMEM((2 * 128, 128), jnp.float32),
            pltpu.VMEM((128,), jnp.float32),
            pltpu.VMEM((128,), jnp.int32),
            pltpu.SemaphoreType.DMA,
            pltpu.SemaphoreType.DMA,
        ],
    )(_pool_body)
    return fn(xyzT, depth, seg, table, prm)


def kernel(roi_boxes3d, gt_boxes3d, rpn_xyz, rpn_features, seg_mask, pts_depth):
    B = roi_boxes3d.shape[0]
    rois_p = jnp.pad(roi_boxes3d, ((0, 0), (0, 0), (0, 1)))
    rois_t = rois_p.transpose(0, 2, 1)
    rois_sel_t, gt_out_t, prm_t = _stage1(rois_p, rois_t, gt_boxes3d)
    rois_sel = rois_sel_t.transpose(0, 2, 1)
    gt_out = gt_out_t.transpose(0, 2, 1)
    prm = prm_t.transpose(0, 2, 1)

    xyzT = rpn_xyz.transpose(0, 2, 1).reshape(-1)
    table = rpn_features.reshape(B * P, 128)

    pts, feats, sd, lab = _stage2(
        xyzT, pts_depth.reshape(-1), seg_mask.reshape(-1), table,
        prm.reshape(-1))
    lab2 = lab.reshape(_NW, 128)
    cls_label = lab2[:, :_RPW].reshape(-1)
    reg_valid = lab2[:, _RPW:2 * _RPW].reshape(-1)

    feats_full = jnp.concatenate(
        [sd.reshape(B * R, NPTS, 2), feats.reshape(B * R, NPTS, 128)],
        axis=-1)
    return (pts.reshape(B * R, NPTS, 3),
            feats_full,
            cls_label,
            reg_valid,
            gt_out[..., :7].reshape(B * R, 7),
            prm[..., 10].reshape(B * R),
            rois_sel[..., :7].reshape(B * R, 7))


# speedup vs baseline: 1.3894x; 1.1608x over previous
import functools

import numpy as np
import jax
import jax.numpy as jnp
from jax import lax
from jax.experimental import pallas as pl
from jax.experimental.pallas import tpu as pltpu
from jax.experimental.pallas import tpu_sc as plsc

M = 512
N = 64
P = 16384
CF = 130
NPTS = 512
FG = 32
BG = 32
R = 64
TWO_PI = float(2 * np.pi)


def _iou_terms(rx, ry_, rz, rh, rw, rl, gx, gy, gz, gh, gw, gl):
    ix1 = jnp.maximum(rx - rl / 2, gx - gl / 2)
    ix2 = jnp.minimum(rx + rl / 2, gx + gl / 2)
    iz1 = jnp.maximum(rz - rw / 2, gz - gw / 2)
    iz2 = jnp.minimum(rz + rw / 2, gz + gw / 2)
    iy1 = jnp.maximum(ry_ - rh, gy - gh)
    iy2 = jnp.minimum(ry_, gy)
    inter = (jnp.clip(ix2 - ix1, 0.0) * jnp.clip(iz2 - iz1, 0.0)
             * jnp.clip(iy2 - iy1, 0.0))
    vol_r = rh * rw * rl
    vol_g = gh * gw * gl
    return inter / jnp.maximum(vol_r + vol_g - inter, 1e-6)


def _select_body(rois_ref, rois_t_ref, gts_ref, rois_out_ref, gt_out_ref,
                 prm_ref):
    r = rois_ref[0]
    rT = rois_t_ref[0]
    g = gts_ref[0]

    def gcol(c):
        return lax.broadcast_in_dim(g[:, c], (M, N), (1,))

    iou = _iou_terms(*[r[:, c:c + 1] for c in range(6)],
                     *[gcol(c) for c in range(6)])
    iouT = _iou_terms(*[rT[c:c + 1] for c in range(6)],
                      *[g[:, c:c + 1] for c in range(6)])

    max_ov = jnp.max(iou, axis=1, keepdims=True)
    maxT = jnp.max(iouT, axis=0, keepdims=True)
    jN = lax.broadcasted_iota(jnp.int32, (M, N), 1)
    gt_asg = jnp.min(jnp.where(iou == max_ov, jN, N), axis=1, keepdims=True)

    vT = jnp.broadcast_to(maxT, (M, M))
    iI = lax.broadcasted_iota(jnp.int32, (M, M), 0)
    jJ = lax.broadcasted_iota(jnp.int32, (M, M), 1)
    before = (vT > max_ov) | ((vT == max_ov) & (jJ < iI))
    rank = jnp.sum(before.astype(jnp.int32), axis=1, keepdims=True)

    slot = jnp.where(rank < FG, rank,
                     jnp.where(rank >= M - BG, rank - (M - R), -1))
    sT = lax.broadcasted_iota(jnp.int32, (M, R), 1)
    O = (slot == sT).astype(jnp.float32)

    dn = (((0,), (0,)), ((), ()))
    hp = lax.Precision.HIGHEST
    rois_sel = lax.dot_general(r, O, dn, precision=hp)
    rois_out_ref[0] = rois_sel

    jNg = lax.broadcasted_iota(jnp.int32, (M, N), 1)
    OG = (gt_asg == jNg).astype(jnp.float32)
    gpick = jnp.dot(OG, g, precision=hp)
    gt_sel = lax.dot_general(gpick, O, dn, precision=hp)
    iou_sel = lax.dot_general(max_ov, O, dn, precision=hp)

    rsel = [rois_sel[c:c + 1] for c in range(7)]
    gsel = [gt_sel[c:c + 1] for c in range(7)]
    cx, cy, cz, h, w, l, ry = rsel
    rym = jnp.mod(ry, TWO_PI)
    cosr = jnp.cos(ry)
    sinr = jnp.sin(ry)
    cosm = jnp.cos(rym)
    sinm = jnp.sin(rym)

    gxl = gsel[0] - cx
    gyl = gsel[1] - cy
    gzl = gsel[2] - cz
    gxn = cosm * gxl - sinm * gzl
    gzn = sinm * gxl + cosm * gzl
    gt_ry_o = gsel[6] - rym
    gt_out_ref[0] = jnp.concatenate(
        [gxn, gyl, gzn, gsel[3], gsel[4], gsel[5], gt_ry_o,
         jnp.zeros((1, R), jnp.float32)], axis=0)

    invalid = (iou_sel > 0.45) & (iou_sel < 0.6)
    clsv = jnp.where(invalid, -1.0, (iou_sel > 0.6).astype(jnp.float32))
    regv = (iou_sel > 0.55).astype(jnp.float32)
    xlim = l / 2 + 1.0
    zlim = w / 2 + 1.0
    ylo = -h - 1.0
    pad = jnp.zeros((5, R), jnp.float32)
    prm_ref[0] = jnp.concatenate(
        [cx, cy, cz, cosr, sinr, xlim, zlim, ylo, clsv, regv, iou_sel, pad],
        axis=0)


def _stage1(rois_p, rois_t, gts_p):
    B = rois_p.shape[0]
    return pl.pallas_call(
        _select_body,
        grid=(B,),
        in_specs=[
            pl.BlockSpec((1, M, 8), lambda b: (b, 0, 0)),
            pl.BlockSpec((1, 8, M), lambda b: (b, 0, 0)),
            pl.BlockSpec((1, N, 8), lambda b: (b, 0, 0)),
        ],
        out_specs=[
            pl.BlockSpec((1, 8, R), lambda b: (b, 0, 0)),
            pl.BlockSpec((1, 8, R), lambda b: (b, 0, 0)),
            pl.BlockSpec((1, 16, R), lambda b: (b, 0, 0)),
        ],
        out_shape=[
            jax.ShapeDtypeStruct((B, 8, R), jnp.float32),
            jax.ShapeDtypeStruct((B, 8, R), jnp.float32),
            jax.ShapeDtypeStruct((B, 16, R), jnp.float32),
        ],
    )(rois_p, rois_t, gts_p)


_info = plsc.get_sparse_core_info()
_NC, _NS, _L = _info.num_cores, _info.num_subcores, _info.num_lanes
_NW = _NC * _NS
_RPW = (4 * R) // _NW
_WPB = R // _RPW
_NSTEP = P // 16
_NCH = NPTS // 128


def _pool_body(xyzT, depth, seg, table, prm_hbm,
               pts_out, feat_out, sd_out, lab_out,
               Xb, Yb, Zb, Db, Sb, bufA, bufB, idxb, ptst, sdst, fst, prmv,
               labacc, sem1, sem2):
    wid = lax.axis_index("s") * _NC + lax.axis_index("c")
    b = wid // _WPB
    grp = wid % _WPB
    bP = b * P

    pltpu.sync_copy(xyzT.at[pl.ds((b * 3 + 0) * P, P)], Xb)
    pltpu.sync_copy(xyzT.at[pl.ds((b * 3 + 1) * P, P)], Yb)
    pltpu.sync_copy(xyzT.at[pl.ds((b * 3 + 2) * P, P)], Zb)
    pltpu.sync_copy(depth.at[pl.ds(bP, P)], Db)
    pltpu.sync_copy(seg.at[pl.ds(bP, P)], Sb)
    pltpu.sync_copy(prm_hbm.at[pl.ds(wid * 128, 128)], prmv)

    lanes = lax.iota(jnp.int32, 16)

    def roi_body(k, carry):
        rb = grp * _RPW + k
        gidx = b * R + rb
        pv = prmv[pl.ds(k * 16, 16)]

        def getp(j):
            return jnp.sum(jnp.where(lanes == j, pv, 0.0))

        cx = getp(0)
        cy = getp(1)
        cz = getp(2)
        cosr = getp(3)
        sinr = getp(4)
        xlim = getp(5)
        zlim = getp(6)
        ylo = getp(7)
        clsv = getp(8)
        regv = getp(9)
        aco = jnp.abs(cosr)
        asi = jnp.abs(sinr)
        xb = xlim * aco + zlim * asi
        zb = xlim * asi + zlim * aco

        @plsc.parallel_loop(0, P, step=16, unroll=8, carry=jnp.int32(0))
        def n(base, offA):
            xv = Xb[pl.ds(base, 16)]
            yv = Yb[pl.ds(base, 16)]
            zv = Zb[pl.ds(base, 16)]
            lx = xv - cx
            ly = yv - cy
            lz = zv - cz
            xn = cosr * lx - sinr * lz
            zn = sinr * lx + cosr * lz
            m = ((jnp.abs(xn) < xlim) & (jnp.abs(zn) < zlim)
                 & (ly > ylo) & (ly < 1.0))
            pidx = base + lanes
            plsc.store_compressed(bufA.at[pl.ds(jnp.minimum(offA, NPTS), 16)],
                                  pidx, mask=m)
            offB = jnp.minimum(base - offA, NPTS)
            plsc.store_compressed(bufB.at[pl.ds(offB, 16)], pidx,
                                  mask=jnp.logical_not(m))
            return offA + jnp.sum(m.astype(jnp.int32))

        nsel = jnp.minimum(n, NPTS)

        zeros16 = jnp.zeros((16,), jnp.int32)
        for j in range(NPTS // 16):
            p = j * 16 + lanes
            av = plsc.load_gather(bufA, [p])
            bv = plsc.load_gather(bufB, [jnp.maximum(p - nsel, 0)])
            iv = jnp.where(p < nsel, av, bv)
            idxb[pl.ds(j * 16, 16)] = iv + bP
            gx = plsc.load_gather(Xb, [iv])
            gy = plsc.load_gather(Yb, [iv])
            gz = plsc.load_gather(Zb, [iv])
            sv = plsc.load_gather(Sb, [iv])
            dv = plsc.load_gather(Db, [iv])
            lx = gx - cx
            lyy = gy - cy
            lz = gz - cz
            xn = cosr * lx - sinr * lz
            zn = sinr * lx + cosr * lz
            p3 = p * 3
            plsc.store_scatter(ptst, [p3], xn)
            plsc.store_scatter(ptst, [p3 + 1], lyy)
            plsc.store_scatter(ptst, [p3 + 2], zn)
            p2 = p * 2
            plsc.store_scatter(sdst, [p2], sv)
            plsc.store_scatter(sdst, [p2 + 1], dv / 70.0 - 0.5)
        pltpu.sync_copy(ptst, pts_out.at[pl.ds(gidx * (NPTS * 3), NPTS * 3)])
        pltpu.sync_copy(sdst, sd_out.at[pl.ds(gidx * (NPTS * 2), NPTS * 2)])

        def gather(ch, sb):
            return pltpu.async_copy(table.at[idxb.at[pl.ds(ch * 128, 128)]],
                                    fst.at[pl.ds(sb * 128, 128)], sem1)

        def putout(ch, sb):
            return pltpu.async_copy(
                fst.at[pl.ds(sb * 128, 128)],
                feat_out.at[pl.ds(gidx * NPTS + ch * 128, 128)], sem2)

        g0 = gather(0, 0)
        g0.wait()
        c0 = putout(0, 0)
        g1 = gather(1, 1)
        g1.wait()
        c1 = putout(1, 1)
        c0.wait()
        g2 = gather(2, 0)
        g2.wait()
        c2 = putout(2, 0)
        c1.wait()
        g3 = gather(3, 1)
        g3.wait()
        c3 = putout(3, 1)
        c2.wait()
        c3.wait()

        valid = n > 0
        cls_i = jnp.where(valid, clsv.astype(jnp.int32), -1)
        reg_i = jnp.where(valid, regv.astype(jnp.int32), 0)
        lane0 = lanes == 0
        plsc.store_scatter(labacc, [zeros16 + k],
                           jnp.broadcast_to(cls_i, (16,)), mask=lane0)
        plsc.store_scatter(labacc, [zeros16 + (k + _RPW)],
                           jnp.broadcast_to(reg_i, (16,)), mask=lane0)
        return carry

    lax.fori_loop(0, _RPW, roi_body, jnp.int32(0))
    pltpu.sync_copy(labacc, lab_out.at[pl.ds(wid * 128, 128)])


def _stage2(xyzT, depth, seg, table, prm):
    B = depth.shape[0] // P
    nroi = B * R
    mesh = plsc.VectorSubcoreMesh(core_axis_name="c", subcore_axis_name="s")
    fn = functools.partial(
        pl.kernel,
        mesh=mesh,
        compiler_params=pltpu.CompilerParams(needs_layout_passes=False),
        out_type=[
            jax.ShapeDtypeStruct((nroi * NPTS * 3,), jnp.float32),
            jax.ShapeDtypeStruct((nroi * NPTS, 128), jnp.float32),
            jax.ShapeDtypeStruct((nroi * NPTS * 2,), jnp.float32),
            jax.ShapeDtypeStruct((_NW * 128,), jnp.int32),
        ],
        scratch_types=[
            pltpu.VMEM((P,), jnp.float32),
            pltpu.VMEM((P,), jnp.float32),
            pltpu.VMEM((P,), jnp.float32),
            pltpu.VMEM((P,), jnp.float32),
            pltpu.VMEM((P,), jnp.float32),
            pltpu.VMEM((NPTS + 16,), jnp.int32),
            pltpu.VMEM((NPTS + 16,), jnp.int32),
            pltpu.VMEM((NPTS,), jnp.int32),
            pltpu.VMEM((NPTS * 3,), jnp.float32),
            pltpu.VMEM((NPTS * 2,), jnp.float32),
            pltpu.VMEM((2 * 128, 128), jnp.float32),
            pltpu.VMEM((128,), jnp.float32),
            pltpu.VMEM((128,), jnp.int32),
            pltpu.SemaphoreType.DMA,
            pltpu.SemaphoreType.DMA,
        ],
    )(_pool_body)
    return fn(xyzT, depth, seg, table, prm)


def kernel(roi_boxes3d, gt_boxes3d, rpn_xyz, rpn_features, seg_mask, pts_depth):
    B = roi_boxes3d.shape[0]
    rois_p = jnp.pad(roi_boxes3d, ((0, 0), (0, 0), (0, 1)))
    rois_t = rois_p.transpose(0, 2, 1)
    rois_sel_t, gt_out_t, prm_t = _stage1(rois_p, rois_t, gt_boxes3d)
    rois_sel = rois_sel_t.transpose(0, 2, 1)
    gt_out = gt_out_t.transpose(0, 2, 1)
    prm = prm_t.transpose(0, 2, 1)

    xyzT = rpn_xyz.transpose(0, 2, 1).reshape(-1)
    table = rpn_features.reshape(B * P, 128)

    pts, feats, sd, lab = _stage2(
        xyzT, pts_depth.reshape(-1), seg_mask.reshape(-1), table,
        prm.reshape(-1))
    lab2 = lab.reshape(_NW, 128)
    cls_label = lab2[:, :_RPW].reshape(-1)
    reg_valid = lab2[:, _RPW:2 * _RPW].reshape(-1)

    feats_full = jnp.concatenate(
        [sd.reshape(B * R, NPTS, 2), feats.reshape(B * R, NPTS, 128)],
        axis=-1)
    return (pts.reshape(B * R, NPTS, 3),
            feats_full,
            cls_label,
            reg_valid,
            gt_out[..., :7].reshape(B * R, 7),
            prm[..., 10].reshape(B * R),
            rois_sel[..., :7].reshape(B * R, 7))


# speedup vs baseline: 1.4003x; 1.0078x over previous
import functools

import numpy as np
import jax
import jax.numpy as jnp
from jax import lax
from jax.experimental import pallas as pl
from jax.experimental.pallas import tpu as pltpu
from jax.experimental.pallas import tpu_sc as plsc

M = 512
N = 64
P = 16384
CF = 130
NPTS = 512
FG = 32
BG = 32
R = 64
TWO_PI = float(2 * np.pi)


def _iou_terms(rx, ry_, rz, rh, rw, rl, gx, gy, gz, gh, gw, gl):
    ix1 = jnp.maximum(rx - rl / 2, gx - gl / 2)
    ix2 = jnp.minimum(rx + rl / 2, gx + gl / 2)
    iz1 = jnp.maximum(rz - rw / 2, gz - gw / 2)
    iz2 = jnp.minimum(rz + rw / 2, gz + gw / 2)
    iy1 = jnp.maximum(ry_ - rh, gy - gh)
    iy2 = jnp.minimum(ry_, gy)
    inter = (jnp.clip(ix2 - ix1, 0.0) * jnp.clip(iz2 - iz1, 0.0)
             * jnp.clip(iy2 - iy1, 0.0))
    vol_r = rh * rw * rl
    vol_g = gh * gw * gl
    return inter / jnp.maximum(vol_r + vol_g - inter, 1e-6)


def _select_body(rois_ref, rois_t_ref, gts_ref, rois_out_ref, gt_out_ref,
                 prm_ref):
    r = rois_ref[0]
    rT = rois_t_ref[0]
    g = gts_ref[0]

    def gcol(c):
        return lax.broadcast_in_dim(g[:, c], (M, N), (1,))

    iou = _iou_terms(*[r[:, c:c + 1] for c in range(6)],
                     *[gcol(c) for c in range(6)])
    iouT = _iou_terms(*[rT[c:c + 1] for c in range(6)],
                      *[g[:, c:c + 1] for c in range(6)])

    max_ov = jnp.max(iou, axis=1, keepdims=True)
    maxT = jnp.max(iouT, axis=0, keepdims=True)
    jN = lax.broadcasted_iota(jnp.int32, (M, N), 1)
    gt_asg = jnp.min(jnp.where(iou == max_ov, jN, N), axis=1, keepdims=True)

    vT = jnp.broadcast_to(maxT, (M, M))
    iI = lax.broadcasted_iota(jnp.int32, (M, M), 0)
    jJ = lax.broadcasted_iota(jnp.int32, (M, M), 1)
    before = (vT > max_ov) | ((vT == max_ov) & (jJ < iI))
    rank = jnp.sum(before.astype(jnp.int32), axis=1, keepdims=True)

    slot = jnp.where(rank < FG, rank,
                     jnp.where(rank >= M - BG, rank - (M - R), -1))
    sT = lax.broadcasted_iota(jnp.int32, (M, R), 1)
    O = (slot == sT).astype(jnp.float32)

    dn = (((0,), (0,)), ((), ()))
    hp = lax.Precision.HIGHEST
    rois_sel = lax.dot_general(r, O, dn, precision=hp)
    rois_out_ref[0] = rois_sel

    jNg = lax.broadcasted_iota(jnp.int32, (M, N), 1)
    OG = (gt_asg == jNg).astype(jnp.float32)
    gpick = jnp.dot(OG, g, precision=hp)
    gt_sel = lax.dot_general(gpick, O, dn, precision=hp)
    iou_sel = lax.dot_general(max_ov, O, dn, precision=hp)

    rsel = [rois_sel[c:c + 1] for c in range(7)]
    gsel = [gt_sel[c:c + 1] for c in range(7)]
    cx, cy, cz, h, w, l, ry = rsel
    rym = jnp.mod(ry, TWO_PI)
    cosr = jnp.cos(ry)
    sinr = jnp.sin(ry)
    cosm = jnp.cos(rym)
    sinm = jnp.sin(rym)

    gxl = gsel[0] - cx
    gyl = gsel[1] - cy
    gzl = gsel[2] - cz
    gxn = cosm * gxl - sinm * gzl
    gzn = sinm * gxl + cosm * gzl
    gt_ry_o = gsel[6] - rym
    gt_out_ref[0] = jnp.concatenate(
        [gxn, gyl, gzn, gsel[3], gsel[4], gsel[5], gt_ry_o,
         jnp.zeros((1, R), jnp.float32)], axis=0)

    invalid = (iou_sel > 0.45) & (iou_sel < 0.6)
    clsv = jnp.where(invalid, -1.0, (iou_sel > 0.6).astype(jnp.float32))
    regv = (iou_sel > 0.55).astype(jnp.float32)
    xlim = l / 2 + 1.0
    zlim = w / 2 + 1.0
    ylo = -h - 1.0
    pad = jnp.zeros((5, R), jnp.float32)
    prm_ref[0] = jnp.concatenate(
        [cx, cy, cz, cosr, sinr, xlim, zlim, ylo, clsv, regv, iou_sel, pad],
        axis=0)


def _stage1(rois_p, rois_t, gts_p):
    B = rois_p.shape[0]
    return pl.pallas_call(
        _select_body,
        grid=(B,),
        in_specs=[
            pl.BlockSpec((1, M, 8), lambda b: (b, 0, 0)),
            pl.BlockSpec((1, 8, M), lambda b: (b, 0, 0)),
            pl.BlockSpec((1, N, 8), lambda b: (b, 0, 0)),
        ],
        out_specs=[
            pl.BlockSpec((1, 8, R), lambda b: (b, 0, 0)),
            pl.BlockSpec((1, 8, R), lambda b: (b, 0, 0)),
            pl.BlockSpec((1, 16, R), lambda b: (b, 0, 0)),
        ],
        out_shape=[
            jax.ShapeDtypeStruct((B, 8, R), jnp.float32),
            jax.ShapeDtypeStruct((B, 8, R), jnp.float32),
            jax.ShapeDtypeStruct((B, 16, R), jnp.float32),
        ],
    )(rois_p, rois_t, gts_p)


_info = plsc.get_sparse_core_info()
_NC, _NS, _L = _info.num_cores, _info.num_subcores, _info.num_lanes
_NW = _NC * _NS
_RPW = (4 * R) // _NW
_WPB = R // _RPW
_NSTEP = P // 16
_NCH = NPTS // 128


def _pool_body(xyzT, depth, seg, table, prm_hbm,
               pts_out, feat_out, sd_out, lab_out,
               Xb, Yb, Zb, Db, Sb, bufA, bufB, idxb, ptst, sdst, fst, prmv,
               labacc, sem1, sem2, sem3):
    wid = lax.axis_index("s") * _NC + lax.axis_index("c")
    b = wid // _WPB
    grp = wid % _WPB
    bP = b * P

    pltpu.sync_copy(xyzT.at[pl.ds((b * 3 + 0) * P, P)], Xb)
    pltpu.sync_copy(xyzT.at[pl.ds((b * 3 + 1) * P, P)], Yb)
    pltpu.sync_copy(xyzT.at[pl.ds((b * 3 + 2) * P, P)], Zb)
    pltpu.sync_copy(depth.at[pl.ds(bP, P)], Db)
    pltpu.sync_copy(seg.at[pl.ds(bP, P)], Sb)
    pltpu.sync_copy(prm_hbm.at[pl.ds(wid * 128, 128)], prmv)

    lanes = lax.iota(jnp.int32, 16)

    def roi_body(k, carry):
        rb = grp * _RPW + k
        gidx = b * R + rb
        pv = prmv[pl.ds(k * 16, 16)]

        def getp(j):
            return jnp.sum(jnp.where(lanes == j, pv, 0.0))

        cx = getp(0)
        cy = getp(1)
        cz = getp(2)
        cosr = getp(3)
        sinr = getp(4)
        xlim = getp(5)
        zlim = getp(6)
        ylo = getp(7)
        clsv = getp(8)
        regv = getp(9)
        aco = jnp.abs(cosr)
        asi = jnp.abs(sinr)
        xb = xlim * aco + zlim * asi
        zb = xlim * asi + zlim * aco

        @plsc.parallel_loop(0, P, step=16, unroll=16, carry=jnp.int32(0))
        def n(base, offA):
            xv = Xb[pl.ds(base, 16)]
            yv = Yb[pl.ds(base, 16)]
            zv = Zb[pl.ds(base, 16)]
            lx = xv - cx
            ly = yv - cy
            lz = zv - cz
            xn = cosr * lx - sinr * lz
            zn = sinr * lx + cosr * lz
            m = ((jnp.abs(xn) < xlim) & (jnp.abs(zn) < zlim)
                 & (ly > ylo) & (ly < 1.0))
            pidx = base + lanes
            plsc.store_compressed(bufA.at[pl.ds(jnp.minimum(offA, NPTS), 16)],
                                  pidx, mask=m)
            offB = jnp.minimum(base - offA, NPTS)
            plsc.store_compressed(bufB.at[pl.ds(offB, 16)], pidx,
                                  mask=jnp.logical_not(m))
            return offA + jnp.sum(m.astype(jnp.int32))

        nsel = jnp.minimum(n, NPTS)

        zeros16 = jnp.zeros((16,), jnp.int32)

        @plsc.parallel_loop(0, NPTS, step=16, unroll=8)
        def _build(p0):
            p = p0 + lanes
            av = plsc.load_gather(bufA, [p])
            bv = plsc.load_gather(bufB, [jnp.maximum(p - nsel, 0)])
            iv = jnp.where(p < nsel, av, bv)
            idxb[pl.ds(p0, 16)] = iv + bP
            gx = plsc.load_gather(Xb, [iv])
            gy = plsc.load_gather(Yb, [iv])
            gz = plsc.load_gather(Zb, [iv])
            sv = plsc.load_gather(Sb, [iv])
            dv = plsc.load_gather(Db, [iv])
            lx = gx - cx
            lyy = gy - cy
            lz = gz - cz
            xn = cosr * lx - sinr * lz
            zn = sinr * lx + cosr * lz
            p3 = p * 3
            plsc.store_scatter(ptst, [p3], xn)
            plsc.store_scatter(ptst, [p3 + 1], lyy)
            plsc.store_scatter(ptst, [p3 + 2], zn)
            p2 = p * 2
            plsc.store_scatter(sdst, [p2], sv)
            plsc.store_scatter(sdst, [p2 + 1], dv / 70.0 - 0.5)
        cp_p = pltpu.async_copy(
            ptst, pts_out.at[pl.ds(gidx * (NPTS * 3), NPTS * 3)], sem3)
        cp_s = pltpu.async_copy(
            sdst, sd_out.at[pl.ds(gidx * (NPTS * 2), NPTS * 2)], sem3)

        def gather(ch, sb):
            return pltpu.async_copy(table.at[idxb.at[pl.ds(ch * 128, 128)]],
                                    fst.at[pl.ds(sb * 128, 128)], sem1)

        def putout(ch, sb):
            return pltpu.async_copy(
                fst.at[pl.ds(sb * 128, 128)],
                feat_out.at[pl.ds(gidx * NPTS + ch * 128, 128)], sem2)

        g0 = gather(0, 0)
        g0.wait()
        c0 = putout(0, 0)
        g1 = gather(1, 1)
        g1.wait()
        c1 = putout(1, 1)
        c0.wait()
        g2 = gather(2, 0)
        g2.wait()
        c2 = putout(2, 0)
        c1.wait()
        g3 = gather(3, 1)
        g3.wait()
        c3 = putout(3, 1)
        c2.wait()
        c3.wait()
        cp_p.wait()
        cp_s.wait()

        valid = n > 0
        cls_i = jnp.where(valid, clsv.astype(jnp.int32), -1)
        reg_i = jnp.where(valid, regv.astype(jnp.int32), 0)
        lane0 = lanes == 0
        plsc.store_scatter(labacc, [zeros16 + k],
                           jnp.broadcast_to(cls_i, (16,)), mask=lane0)
        plsc.store_scatter(labacc, [zeros16 + (k + _RPW)],
                           jnp.broadcast_to(reg_i, (16,)), mask=lane0)
        return carry

    lax.fori_loop(0, _RPW, roi_body, jnp.int32(0))
    pltpu.sync_copy(labacc, lab_out.at[pl.ds(wid * 128, 128)])


def _stage2(xyzT, depth, seg, table, prm):
    B = depth.shape[0] // P
    nroi = B * R
    mesh = plsc.VectorSubcoreMesh(core_axis_name="c", subcore_axis_name="s")
    fn = functools.partial(
        pl.kernel,
        mesh=mesh,
        compiler_params=pltpu.CompilerParams(needs_layout_passes=False),
        out_type=[
            jax.ShapeDtypeStruct((nroi * NPTS * 3,), jnp.float32),
            jax.ShapeDtypeStruct((nroi * NPTS, 128), jnp.float32),
            jax.ShapeDtypeStruct((nroi * NPTS * 2,), jnp.float32),
            jax.ShapeDtypeStruct((_NW * 128,), jnp.int32),
        ],
        scratch_types=[
            pltpu.VMEM((P,), jnp.float32),
            pltpu.VMEM((P,), jnp.float32),
            pltpu.VMEM((P,), jnp.float32),
            pltpu.VMEM((P,), jnp.float32),
            pltpu.VMEM((P,), jnp.float32),
            pltpu.VMEM((NPTS + 16,), jnp.int32),
            pltpu.VMEM((NPTS + 16,), jnp.int32),
            pltpu.VMEM((NPTS,), jnp.int32),
            pltpu.VMEM((NPTS * 3,), jnp.float32),
            pltpu.VMEM((NPTS * 2,), jnp.float32),
            pltpu.VMEM((2 * 128, 128), jnp.float32),
            pltpu.VMEM((128,), jnp.float32),
            pltpu.VMEM((128,), jnp.int32),
            pltpu.SemaphoreType.DMA,
            pltpu.SemaphoreType.DMA,
            pltpu.SemaphoreType.DMA,
        ],
    )(_pool_body)
    return fn(xyzT, depth, seg, table, prm)


def kernel(roi_boxes3d, gt_boxes3d, rpn_xyz, rpn_features, seg_mask, pts_depth):
    B = roi_boxes3d.shape[0]
    rois_p = jnp.pad(roi_boxes3d, ((0, 0), (0, 0), (0, 1)))
    rois_t = rois_p.transpose(0, 2, 1)
    rois_sel_t, gt_out_t, prm_t = _stage1(rois_p, rois_t, gt_boxes3d)
    rois_sel = rois_sel_t.transpose(0, 2, 1)
    gt_out = gt_out_t.transpose(0, 2, 1)
    prm = prm_t.transpose(0, 2, 1)

    xyzT = rpn_xyz.transpose(0, 2, 1).reshape(-1)
    table = rpn_features.reshape(B * P, 128)

    pts, feats, sd, lab = _stage2(
        xyzT, pts_depth.reshape(-1), seg_mask.reshape(-1), table,
        prm.reshape(-1))
    lab2 = lab.reshape(_NW, 128)
    cls_label = lab2[:, :_RPW].reshape(-1)
    reg_valid = lab2[:, _RPW:2 * _RPW].reshape(-1)

    feats_full = jnp.concatenate(
        [sd.reshape(B * R, NPTS, 2), feats.reshape(B * R, NPTS, 128)],
        axis=-1)
    return (pts.reshape(B * R, NPTS, 3),
            feats_full,
            cls_label,
            reg_valid,
            gt_out[..., :7].reshape(B * R, 7),
            prm[..., 10].reshape(B * R),
            rois_sel[..., :7].reshape(B * R, 7))
